# baseline (speedup 1.0000x reference)
import functools

import jax
import jax.numpy as jnp
from jax import lax
from jax.experimental import pallas as pl
from jax.experimental.pallas import tpu as pltpu
from jax.experimental.pallas import tpu_sc as plsc

_CAT = 80
_DIM = 256
_N = 100000
_T = 0.07
_LOSS_WEIGHT = 1.0

_NC = 1
_NS = 16
_NW = _NC * _NS
_L = 16
_B = 2000
_G_TC = _N // _B

_CCH = 800
_NCH = _N // _CCH
_BASE_ITERS = _NCH // _NW
_EXTRA = _NCH - _BASE_ITERS * _NW


def _zero(ref, nwords):
    z16 = jnp.zeros((_L,), jnp.float32)
    unroll = 8

    def body(k, carry):
        for u in range(unroll):
            ref[pl.ds((k * unroll + u) * _L, _L)] = z16
        return carry

    lax.fori_loop(0, nwords // (_L * unroll), body, 0)


def _count_chunk(idx_b, cnt):
    ones16 = jnp.ones((_L,), jnp.float32)

    @plsc.parallel_loop(0, _CCH // _L)
    def group(g):
        t16 = idx_b[pl.ds(g * _L, _L)]
        for l in range(_L):
            t = t16[l]
            plsc.addupdate(cnt.at[pl.ds(t * _NS, _NS)], ones16)


def _sc_body(targets, out_c, idx0, idx1, cnt, sem_i0, sem_i1):
    cid = lax.axis_index("c")
    sid = lax.axis_index("s")
    w = sid * _NC + cid

    _zero(cnt, _CAT * _NS)

    niter = jnp.where(w < _EXTRA, _BASE_ITERS + 1, _BASE_ITERS)
    q0 = w * _BASE_ITERS + jnp.minimum(w, _EXTRA)

    @pl.when(niter > 0)
    def _():
        pltpu.async_copy(targets.at[pl.ds(q0 * _CCH, _CCH)], idx0, sem_i0)

        def body(i, carry):
            nxt = i + 1
            nbase = (q0 + nxt) * _CCH
            nxt_even = (nxt % 2) == 0
            do_next = nxt < niter

            @pl.when(do_next & nxt_even)
            def _():
                pltpu.async_copy(targets.at[pl.ds(nbase, _CCH)], idx0, sem_i0)

            @pl.when(do_next & jnp.logical_not(nxt_even))
            def _():
                pltpu.async_copy(targets.at[pl.ds(nbase, _CCH)], idx1, sem_i1)

            cur_even = (i % 2) == 0

            @pl.when(cur_even)
            def _():
                pltpu.make_async_copy(
                    targets.at[pl.ds(0, _CCH)], idx0, sem_i0).wait()
                _count_chunk(idx0, cnt)

            @pl.when(jnp.logical_not(cur_even))
            def _():
                pltpu.make_async_copy(
                    targets.at[pl.ds(0, _CCH)], idx1, sem_i1).wait()
                _count_chunk(idx1, cnt)

            return carry

        lax.fori_loop(0, niter, body, 0)

    pltpu.sync_copy(cnt, out_c.at[w])


def _tc_matmul_body(tgt_ref, feats_ref, out_s):
    k = pl.program_id(0)
    cls_iota = lax.broadcasted_iota(jnp.int32, (_CAT, _B), 0)
    oh = (tgt_ref[0] == cls_iota).astype(jnp.float32)
    part = lax.dot_general(
        oh, feats_ref[...], (((1,), (0,)), ((), ())),
        preferred_element_type=jnp.float32)

    @pl.when(k == 0)
    def _():
        out_s[...] = part

    @pl.when(k > 0)
    def _():
        out_s[...] += part


def _tc_loss_body(sums_ref, cnts_ref, protos_ref, out_ref):
    sums = sums_ref[...]
    csum = jnp.sum(cnts_ref[...], axis=0, keepdims=True)
    jj = lax.broadcasted_iota(jnp.int32, (_CAT, _CAT * _NS), 1)
    ii = lax.broadcasted_iota(jnp.int32, (_CAT, _CAT * _NS), 0)
    sel = (jj == ii * _NS).astype(jnp.float32)
    counts = lax.dot_general(
        sel, csum, (((1,), (1,)), ((), ())),
        preferred_element_type=jnp.float32)
    present = counts > 0.0

    means = sums / jnp.maximum(counts, 1.0)
    delta = jnp.where(present, means, jnp.float32(0.01))

    protos = protos_ref[...]
    v1 = protos / jnp.sqrt(jnp.sum(protos * protos, axis=1, keepdims=True))
    v2 = delta / jnp.sqrt(jnp.sum(delta * delta, axis=1, keepdims=True))

    logits = lax.dot_general(
        v1, v2, (((1,), (1,)), ((), ())),
        preferred_element_type=jnp.float32) / _T
    p = jax.nn.sigmoid(logits)
    logp = jnp.maximum(jnp.log(p), -100.0)
    log1mp = jnp.maximum(jnp.log(1.0 - p), -100.0)

    r = lax.broadcasted_iota(jnp.int32, (_CAT, _CAT), 0)
    c = lax.broadcasted_iota(jnp.int32, (_CAT, _CAT), 1)
    eye = r == c
    lossm = jnp.where(eye, -logp, -log1mp)

    n_present = jnp.sum(present.astype(jnp.float32))
    diag_scale = 1.0 / (2.0 * n_present)
    off_scale = 1.0 / (2.0 * n_present * (_CAT - 1))
    scaled = jnp.where(eye, lossm * diag_scale, lossm * off_scale)

    col_sum = jnp.sum(scaled, axis=0)
    present_cols = (counts[:, 0] > 0.0).astype(jnp.float32)
    total = jnp.sum(col_sum * present_cols) * _LOSS_WEIGHT
    out_ref[...] = total * jnp.ones((1, 1), jnp.float32)


@jax.jit
def kernel(cls_feats, cls_targets, prototypes):
    mesh = plsc.VectorSubcoreMesh(
        core_axis_name="c", subcore_axis_name="s", num_cores=_NC)
    sc_call = pl.kernel(
        _sc_body,
        out_type=jax.ShapeDtypeStruct((_NW, _CAT * _NS), jnp.float32),
        mesh=mesh,
        scratch_types=[
            pltpu.VMEM((_CCH,), jnp.int32),
            pltpu.VMEM((_CCH,), jnp.int32),
            pltpu.VMEM((_CAT * _NS,), jnp.float32),
            pltpu.SemaphoreType.DMA,
            pltpu.SemaphoreType.DMA,
        ],
        compiler_params=pltpu.CompilerParams(needs_layout_passes=False),
    )
    part_cnts = sc_call(cls_targets)

    tgt2d = jnp.reshape(cls_targets, (_N // _B, 1, _B))
    tc_sums = pl.pallas_call(
        _tc_matmul_body,
        grid=(_G_TC,),
        in_specs=[
            pl.BlockSpec((1, 1, _B), lambda k: (k, 0, 0)),
            pl.BlockSpec((_B, _DIM), lambda k: (k, 0)),
        ],
        out_specs=pl.BlockSpec((_CAT, _DIM), lambda k: (0, 0)),
        out_shape=jax.ShapeDtypeStruct((_CAT, _DIM), jnp.float32),
    )(tgt2d, cls_feats)

    loss = pl.pallas_call(
        _tc_loss_body,
        out_shape=jax.ShapeDtypeStruct((1, 1), jnp.float32),
    )(tc_sums, part_cnts, prototypes)
    return loss[0, 0]


# speedup vs baseline: 1.2630x; 1.2630x over previous
import functools

import jax
import jax.numpy as jnp
from jax import lax
from jax.experimental import pallas as pl
from jax.experimental.pallas import tpu as pltpu
from jax.experimental.pallas import tpu_sc as plsc

_CAT = 80
_DIM = 256
_N = 100000
_T = 0.07
_LOSS_WEIGHT = 1.0

_NC = 1
_NS = 16
_NW = _NC * _NS
_L = 16
_B = 4000
_G_TC = _N // _B

_CCH = 800
_NCH = _N // _CCH
_BASE_ITERS = _NCH // _NW
_EXTRA = _NCH - _BASE_ITERS * _NW


def _zero(ref, nwords):
    z16 = jnp.zeros((_L,), jnp.float32)
    unroll = 8

    def body(k, carry):
        for u in range(unroll):
            ref[pl.ds((k * unroll + u) * _L, _L)] = z16
        return carry

    lax.fori_loop(0, nwords // (_L * unroll), body, 0)


def _count_chunk(idx_b, cnt):
    ones16 = jnp.ones((_L,), jnp.float32)

    @plsc.parallel_loop(0, _CCH // _L)
    def group(g):
        t16 = idx_b[pl.ds(g * _L, _L)]
        for l in range(_L):
            t = t16[l]
            plsc.addupdate(cnt.at[pl.ds(t * _NS, _NS)], ones16)


def _sc_body(targets, out_c, idx0, idx1, cnt, sem_i0, sem_i1):
    cid = lax.axis_index("c")
    sid = lax.axis_index("s")
    w = sid * _NC + cid

    _zero(cnt, _CAT * _NS)

    niter = jnp.where(w < _EXTRA, _BASE_ITERS + 1, _BASE_ITERS)
    q0 = w * _BASE_ITERS + jnp.minimum(w, _EXTRA)

    @pl.when(niter > 0)
    def _():
        pltpu.async_copy(targets.at[pl.ds(q0 * _CCH, _CCH)], idx0, sem_i0)

        def body(i, carry):
            nxt = i + 1
            nbase = (q0 + nxt) * _CCH
            nxt_even = (nxt % 2) == 0
            do_next = nxt < niter

            @pl.when(do_next & nxt_even)
            def _():
                pltpu.async_copy(targets.at[pl.ds(nbase, _CCH)], idx0, sem_i0)

            @pl.when(do_next & jnp.logical_not(nxt_even))
            def _():
                pltpu.async_copy(targets.at[pl.ds(nbase, _CCH)], idx1, sem_i1)

            cur_even = (i % 2) == 0

            @pl.when(cur_even)
            def _():
                pltpu.make_async_copy(
                    targets.at[pl.ds(0, _CCH)], idx0, sem_i0).wait()
                _count_chunk(idx0, cnt)

            @pl.when(jnp.logical_not(cur_even))
            def _():
                pltpu.make_async_copy(
                    targets.at[pl.ds(0, _CCH)], idx1, sem_i1).wait()
                _count_chunk(idx1, cnt)

            return carry

        lax.fori_loop(0, niter, body, 0)

    pltpu.sync_copy(cnt, out_c.at[w])


def _tc_matmul_body(tgt_ref, feats_ref, out_s):
    k = pl.program_id(0)
    cls_iota = lax.broadcasted_iota(jnp.int32, (_CAT, _B), 0)
    oh = (tgt_ref[0] == cls_iota).astype(jnp.float32)
    part = lax.dot_general(
        oh, feats_ref[...], (((1,), (0,)), ((), ())),
        preferred_element_type=jnp.float32)

    @pl.when(k == 0)
    def _():
        out_s[...] = part

    @pl.when(k > 0)
    def _():
        out_s[...] += part


def _tc_loss_body(sums_ref, cnts_ref, protos_ref, out_ref):
    sums = sums_ref[...]
    csum = jnp.sum(cnts_ref[...], axis=0, keepdims=True)
    jj = lax.broadcasted_iota(jnp.int32, (_CAT, _CAT * _NS), 1)
    ii = lax.broadcasted_iota(jnp.int32, (_CAT, _CAT * _NS), 0)
    sel = (jj == ii * _NS).astype(jnp.float32)
    counts = lax.dot_general(
        sel, csum, (((1,), (1,)), ((), ())),
        preferred_element_type=jnp.float32)
    present = counts > 0.0

    means = sums / jnp.maximum(counts, 1.0)
    delta = jnp.where(present, means, jnp.float32(0.01))

    protos = protos_ref[...]
    v1 = protos / jnp.sqrt(jnp.sum(protos * protos, axis=1, keepdims=True))
    v2 = delta / jnp.sqrt(jnp.sum(delta * delta, axis=1, keepdims=True))

    logits = lax.dot_general(
        v1, v2, (((1,), (1,)), ((), ())),
        preferred_element_type=jnp.float32) / _T
    p = jax.nn.sigmoid(logits)
    logp = jnp.maximum(jnp.log(p), -100.0)
    log1mp = jnp.maximum(jnp.log(1.0 - p), -100.0)

    r = lax.broadcasted_iota(jnp.int32, (_CAT, _CAT), 0)
    c = lax.broadcasted_iota(jnp.int32, (_CAT, _CAT), 1)
    eye = r == c
    lossm = jnp.where(eye, -logp, -log1mp)

    n_present = jnp.sum(present.astype(jnp.float32))
    diag_scale = 1.0 / (2.0 * n_present)
    off_scale = 1.0 / (2.0 * n_present * (_CAT - 1))
    scaled = jnp.where(eye, lossm * diag_scale, lossm * off_scale)

    col_sum = jnp.sum(scaled, axis=0)
    present_cols = (counts[:, 0] > 0.0).astype(jnp.float32)
    total = jnp.sum(col_sum * present_cols) * _LOSS_WEIGHT
    out_ref[...] = total * jnp.ones((1, 1), jnp.float32)


@jax.jit
def kernel(cls_feats, cls_targets, prototypes):
    mesh = plsc.VectorSubcoreMesh(
        core_axis_name="c", subcore_axis_name="s", num_cores=_NC)
    sc_call = pl.kernel(
        _sc_body,
        out_type=jax.ShapeDtypeStruct((_NW, _CAT * _NS), jnp.float32),
        mesh=mesh,
        scratch_types=[
            pltpu.VMEM((_CCH,), jnp.int32),
            pltpu.VMEM((_CCH,), jnp.int32),
            pltpu.VMEM((_CAT * _NS,), jnp.float32),
            pltpu.SemaphoreType.DMA,
            pltpu.SemaphoreType.DMA,
        ],
        compiler_params=pltpu.CompilerParams(needs_layout_passes=False),
    )
    part_cnts = sc_call(cls_targets)

    tgt2d = jnp.reshape(cls_targets, (_N // _B, 1, _B))
    tc_sums = pl.pallas_call(
        _tc_matmul_body,
        grid=(_G_TC,),
        in_specs=[
            pl.BlockSpec((1, 1, _B), lambda k: (k, 0, 0)),
            pl.BlockSpec((_B, _DIM), lambda k: (k, 0)),
        ],
        out_specs=pl.BlockSpec((_CAT, _DIM), lambda k: (0, 0)),
        out_shape=jax.ShapeDtypeStruct((_CAT, _DIM), jnp.float32),
    )(tgt2d, cls_feats)

    loss = pl.pallas_call(
        _tc_loss_body,
        out_shape=jax.ShapeDtypeStruct((1, 1), jnp.float32),
    )(tc_sums, part_cnts, prototypes)
    return loss[0, 0]


# speedup vs baseline: 1.3324x; 1.0550x over previous
import functools

import jax
import jax.numpy as jnp
from jax import lax
from jax.experimental import pallas as pl
from jax.experimental.pallas import tpu as pltpu
from jax.experimental.pallas import tpu_sc as plsc

_CAT = 80
_DIM = 256
_N = 100000
_T = 0.07
_LOSS_WEIGHT = 1.0

_NC = 1
_NS = 16
_NW = _NC * _NS
_L = 16
_B = 5000
_G_TC = _N // _B

_CCH = 800
_NCH = _N // _CCH
_BASE_ITERS = _NCH // _NW
_EXTRA = _NCH - _BASE_ITERS * _NW


def _zero(ref, nwords):
    z16 = jnp.zeros((_L,), jnp.float32)
    unroll = 8

    def body(k, carry):
        for u in range(unroll):
            ref[pl.ds((k * unroll + u) * _L, _L)] = z16
        return carry

    lax.fori_loop(0, nwords // (_L * unroll), body, 0)


def _count_chunk(idx_b, cnt):
    ones16 = jnp.ones((_L,), jnp.float32)

    @plsc.parallel_loop(0, _CCH // _L)
    def group(g):
        t16 = idx_b[pl.ds(g * _L, _L)]
        for l in range(_L):
            t = t16[l]
            plsc.addupdate(cnt.at[pl.ds(t * _NS, _NS)], ones16)


def _sc_body(targets, out_c, idx0, idx1, cnt, sem_i0, sem_i1):
    cid = lax.axis_index("c")
    sid = lax.axis_index("s")
    w = sid * _NC + cid

    _zero(cnt, _CAT * _NS)

    niter = jnp.where(w < _EXTRA, _BASE_ITERS + 1, _BASE_ITERS)
    q0 = w * _BASE_ITERS + jnp.minimum(w, _EXTRA)

    @pl.when(niter > 0)
    def _():
        pltpu.async_copy(targets.at[pl.ds(q0 * _CCH, _CCH)], idx0, sem_i0)

        def body(i, carry):
            nxt = i + 1
            nbase = (q0 + nxt) * _CCH
            nxt_even = (nxt % 2) == 0
            do_next = nxt < niter

            @pl.when(do_next & nxt_even)
            def _():
                pltpu.async_copy(targets.at[pl.ds(nbase, _CCH)], idx0, sem_i0)

            @pl.when(do_next & jnp.logical_not(nxt_even))
            def _():
                pltpu.async_copy(targets.at[pl.ds(nbase, _CCH)], idx1, sem_i1)

            cur_even = (i % 2) == 0

            @pl.when(cur_even)
            def _():
                pltpu.make_async_copy(
                    targets.at[pl.ds(0, _CCH)], idx0, sem_i0).wait()
                _count_chunk(idx0, cnt)

            @pl.when(jnp.logical_not(cur_even))
            def _():
                pltpu.make_async_copy(
                    targets.at[pl.ds(0, _CCH)], idx1, sem_i1).wait()
                _count_chunk(idx1, cnt)

            return carry

        lax.fori_loop(0, niter, body, 0)

    pltpu.sync_copy(cnt, out_c.at[w])


def _tc_matmul_body(tgt_ref, feats_ref, out_s):
    k = pl.program_id(0)
    cls_iota = lax.broadcasted_iota(jnp.int32, (_CAT, _B), 0)
    oh = (tgt_ref[0] == cls_iota).astype(jnp.float32)
    part = lax.dot_general(
        oh, feats_ref[...], (((1,), (0,)), ((), ())),
        preferred_element_type=jnp.float32)

    @pl.when(k == 0)
    def _():
        out_s[...] = part

    @pl.when(k > 0)
    def _():
        out_s[...] += part


def _tc_loss_body(sums_ref, cnts_ref, protos_ref, out_ref):
    sums = sums_ref[...]
    csum = jnp.sum(cnts_ref[...], axis=0, keepdims=True)
    jj = lax.broadcasted_iota(jnp.int32, (_CAT, _CAT * _NS), 1)
    ii = lax.broadcasted_iota(jnp.int32, (_CAT, _CAT * _NS), 0)
    sel = (jj == ii * _NS).astype(jnp.float32)
    counts = lax.dot_general(
        sel, csum, (((1,), (1,)), ((), ())),
        preferred_element_type=jnp.float32)
    present = counts > 0.0

    means = sums / jnp.maximum(counts, 1.0)
    delta = jnp.where(present, means, jnp.float32(0.01))

    protos = protos_ref[...]
    v1 = protos / jnp.sqrt(jnp.sum(protos * protos, axis=1, keepdims=True))
    v2 = delta / jnp.sqrt(jnp.sum(delta * delta, axis=1, keepdims=True))

    logits = lax.dot_general(
        v1, v2, (((1,), (1,)), ((), ())),
        preferred_element_type=jnp.float32) / _T
    p = jax.nn.sigmoid(logits)
    logp = jnp.maximum(jnp.log(p), -100.0)
    log1mp = jnp.maximum(jnp.log(1.0 - p), -100.0)

    r = lax.broadcasted_iota(jnp.int32, (_CAT, _CAT), 0)
    c = lax.broadcasted_iota(jnp.int32, (_CAT, _CAT), 1)
    eye = r == c
    lossm = jnp.where(eye, -logp, -log1mp)

    n_present = jnp.sum(present.astype(jnp.float32))
    diag_scale = 1.0 / (2.0 * n_present)
    off_scale = 1.0 / (2.0 * n_present * (_CAT - 1))
    scaled = jnp.where(eye, lossm * diag_scale, lossm * off_scale)

    col_sum = jnp.sum(scaled, axis=0)
    present_cols = (counts[:, 0] > 0.0).astype(jnp.float32)
    total = jnp.sum(col_sum * present_cols) * _LOSS_WEIGHT
    out_ref[...] = total * jnp.ones((1, 1), jnp.float32)


@jax.jit
def kernel(cls_feats, cls_targets, prototypes):
    mesh = plsc.VectorSubcoreMesh(
        core_axis_name="c", subcore_axis_name="s", num_cores=_NC)
    sc_call = pl.kernel(
        _sc_body,
        out_type=jax.ShapeDtypeStruct((_NW, _CAT * _NS), jnp.float32),
        mesh=mesh,
        scratch_types=[
            pltpu.VMEM((_CCH,), jnp.int32),
            pltpu.VMEM((_CCH,), jnp.int32),
            pltpu.VMEM((_CAT * _NS,), jnp.float32),
            pltpu.SemaphoreType.DMA,
            pltpu.SemaphoreType.DMA,
        ],
        compiler_params=pltpu.CompilerParams(needs_layout_passes=False),
    )
    part_cnts = sc_call(cls_targets)

    tgt2d = jnp.reshape(cls_targets, (_N // _B, 1, _B))
    tc_sums = pl.pallas_call(
        _tc_matmul_body,
        grid=(_G_TC,),
        in_specs=[
            pl.BlockSpec((1, 1, _B), lambda k: (k, 0, 0)),
            pl.BlockSpec((_B, _DIM), lambda k: (k, 0)),
        ],
        out_specs=pl.BlockSpec((_CAT, _DIM), lambda k: (0, 0)),
        out_shape=jax.ShapeDtypeStruct((_CAT, _DIM), jnp.float32),
    )(tgt2d, cls_feats)

    loss = pl.pallas_call(
        _tc_loss_body,
        out_shape=jax.ShapeDtypeStruct((1, 1), jnp.float32),
    )(tc_sums, part_cnts, prototypes)
    return loss[0, 0]


# speedup vs baseline: 1.3738x; 1.0311x over previous
import functools

import jax
import jax.numpy as jnp
from jax import lax
from jax.experimental import pallas as pl
from jax.experimental.pallas import tpu as pltpu
from jax.experimental.pallas import tpu_sc as plsc

_CAT = 80
_DIM = 256
_N = 100000
_T = 0.07
_LOSS_WEIGHT = 1.0

_NC = 1
_NS = 16
_NW = _NC * _NS
_L = 16
_B = 10000
_G_TC = _N // _B

_CCH = 800
_NCH = _N // _CCH
_BASE_ITERS = _NCH // _NW
_EXTRA = _NCH - _BASE_ITERS * _NW


def _zero(ref, nwords):
    z16 = jnp.zeros((_L,), jnp.float32)
    unroll = 8

    def body(k, carry):
        for u in range(unroll):
            ref[pl.ds((k * unroll + u) * _L, _L)] = z16
        return carry

    lax.fori_loop(0, nwords // (_L * unroll), body, 0)


def _count_chunk(idx_b, cnt):
    ones16 = jnp.ones((_L,), jnp.float32)

    @plsc.parallel_loop(0, _CCH // _L)
    def group(g):
        t16 = idx_b[pl.ds(g * _L, _L)]
        for l in range(_L):
            t = t16[l]
            plsc.addupdate(cnt.at[pl.ds(t * _NS, _NS)], ones16)


def _sc_body(targets, out_c, idx0, idx1, cnt, sem_i0, sem_i1):
    cid = lax.axis_index("c")
    sid = lax.axis_index("s")
    w = sid * _NC + cid

    _zero(cnt, _CAT * _NS)

    niter = jnp.where(w < _EXTRA, _BASE_ITERS + 1, _BASE_ITERS)
    q0 = w * _BASE_ITERS + jnp.minimum(w, _EXTRA)

    @pl.when(niter > 0)
    def _():
        pltpu.async_copy(targets.at[pl.ds(q0 * _CCH, _CCH)], idx0, sem_i0)

        def body(i, carry):
            nxt = i + 1
            nbase = (q0 + nxt) * _CCH
            nxt_even = (nxt % 2) == 0
            do_next = nxt < niter

            @pl.when(do_next & nxt_even)
            def _():
                pltpu.async_copy(targets.at[pl.ds(nbase, _CCH)], idx0, sem_i0)

            @pl.when(do_next & jnp.logical_not(nxt_even))
            def _():
                pltpu.async_copy(targets.at[pl.ds(nbase, _CCH)], idx1, sem_i1)

            cur_even = (i % 2) == 0

            @pl.when(cur_even)
            def _():
                pltpu.make_async_copy(
                    targets.at[pl.ds(0, _CCH)], idx0, sem_i0).wait()
                _count_chunk(idx0, cnt)

            @pl.when(jnp.logical_not(cur_even))
            def _():
                pltpu.make_async_copy(
                    targets.at[pl.ds(0, _CCH)], idx1, sem_i1).wait()
                _count_chunk(idx1, cnt)

            return carry

        lax.fori_loop(0, niter, body, 0)

    pltpu.sync_copy(cnt, out_c.at[w])


def _tc_matmul_body(tgt_ref, feats_ref, out_s):
    k = pl.program_id(0)
    cls_iota = lax.broadcasted_iota(jnp.int32, (_CAT, _B), 0)
    oh = (tgt_ref[0] == cls_iota).astype(jnp.float32)
    part = lax.dot_general(
        oh, feats_ref[...], (((1,), (0,)), ((), ())),
        preferred_element_type=jnp.float32)

    @pl.when(k == 0)
    def _():
        out_s[...] = part

    @pl.when(k > 0)
    def _():
        out_s[...] += part


def _tc_loss_body(sums_ref, cnts_ref, protos_ref, out_ref):
    sums = sums_ref[...]
    csum = jnp.sum(cnts_ref[...], axis=0, keepdims=True)
    jj = lax.broadcasted_iota(jnp.int32, (_CAT, _CAT * _NS), 1)
    ii = lax.broadcasted_iota(jnp.int32, (_CAT, _CAT * _NS), 0)
    sel = (jj == ii * _NS).astype(jnp.float32)
    counts = lax.dot_general(
        sel, csum, (((1,), (1,)), ((), ())),
        preferred_element_type=jnp.float32)
    present = counts > 0.0

    means = sums / jnp.maximum(counts, 1.0)
    delta = jnp.where(present, means, jnp.float32(0.01))

    protos = protos_ref[...]
    v1 = protos / jnp.sqrt(jnp.sum(protos * protos, axis=1, keepdims=True))
    v2 = delta / jnp.sqrt(jnp.sum(delta * delta, axis=1, keepdims=True))

    logits = lax.dot_general(
        v1, v2, (((1,), (1,)), ((), ())),
        preferred_element_type=jnp.float32) / _T
    p = jax.nn.sigmoid(logits)
    logp = jnp.maximum(jnp.log(p), -100.0)
    log1mp = jnp.maximum(jnp.log(1.0 - p), -100.0)

    r = lax.broadcasted_iota(jnp.int32, (_CAT, _CAT), 0)
    c = lax.broadcasted_iota(jnp.int32, (_CAT, _CAT), 1)
    eye = r == c
    lossm = jnp.where(eye, -logp, -log1mp)

    n_present = jnp.sum(present.astype(jnp.float32))
    diag_scale = 1.0 / (2.0 * n_present)
    off_scale = 1.0 / (2.0 * n_present * (_CAT - 1))
    scaled = jnp.where(eye, lossm * diag_scale, lossm * off_scale)

    col_sum = jnp.sum(scaled, axis=0)
    present_cols = (counts[:, 0] > 0.0).astype(jnp.float32)
    total = jnp.sum(col_sum * present_cols) * _LOSS_WEIGHT
    out_ref[...] = total * jnp.ones((1, 1), jnp.float32)


@jax.jit
def kernel(cls_feats, cls_targets, prototypes):
    mesh = plsc.VectorSubcoreMesh(
        core_axis_name="c", subcore_axis_name="s", num_cores=_NC)
    sc_call = pl.kernel(
        _sc_body,
        out_type=jax.ShapeDtypeStruct((_NW, _CAT * _NS), jnp.float32),
        mesh=mesh,
        scratch_types=[
            pltpu.VMEM((_CCH,), jnp.int32),
            pltpu.VMEM((_CCH,), jnp.int32),
            pltpu.VMEM((_CAT * _NS,), jnp.float32),
            pltpu.SemaphoreType.DMA,
            pltpu.SemaphoreType.DMA,
        ],
        compiler_params=pltpu.CompilerParams(needs_layout_passes=False),
    )
    part_cnts = sc_call(cls_targets)

    tgt2d = jnp.reshape(cls_targets, (_N // _B, 1, _B))
    tc_sums = pl.pallas_call(
        _tc_matmul_body,
        grid=(_G_TC,),
        in_specs=[
            pl.BlockSpec((1, 1, _B), lambda k: (k, 0, 0)),
            pl.BlockSpec((_B, _DIM), lambda k: (k, 0)),
        ],
        out_specs=pl.BlockSpec((_CAT, _DIM), lambda k: (0, 0)),
        out_shape=jax.ShapeDtypeStruct((_CAT, _DIM), jnp.float32),
    )(tgt2d, cls_feats)

    loss = pl.pallas_call(
        _tc_loss_body,
        out_shape=jax.ShapeDtypeStruct((1, 1), jnp.float32),
    )(tc_sums, part_cnts, prototypes)
    return loss[0, 0]
